# baseline (speedup 1.0000x reference)
import functools

import jax
import jax.numpy as jnp
from jax import lax
from jax.experimental import pallas as pl
from jax.experimental.pallas import tpu as pltpu
from jax.experimental.pallas import tpu_sc as plsc

NUM_BLOCKS = 8
M = 2000
PM = 2048
D = 2048
NC = 2
NS = 16
W_FULL = 504
W_TAIL = 488
C = 24
NBUF = 2
ITERS = -(-W_FULL // C)


def _unpad(inp):
    mesh = plsc.VectorSubcoreMesh(core_axis_name="c", subcore_axis_name="s")

    @functools.partial(
        pl.kernel,
        mesh=mesh,
        out_type=jax.ShapeDtypeStruct((NUM_BLOCKS * M, D), jnp.float32),
        scratch_types=(
            [pltpu.VMEM((C, D), jnp.float32)] * NBUF
            + [pltpu.SemaphoreType.DMA] * (2 * NBUF)
        ),
    )
    def k(inp_hbm, out_hbm, *scr):
        bufs = scr[:NBUF]
        isems = scr[NBUF : 2 * NBUF]
        osems = scr[2 * NBUF :]
        wid = lax.axis_index("s") * NC + lax.axis_index("c")
        blk = wid // 4
        sub = wid % 4
        off = sub * W_FULL
        src0 = blk * PM + off
        dst0 = blk * M + off
        is_tail = sub == 3

        def base(i):
            bf = min(i * C, W_FULL - C)
            bt = min(i * C, W_TAIL - C)
            if bf == bt:
                return bf
            return jnp.where(is_tail, bt, bf)

        def start_in(i):
            slot = i % NBUF
            s = pl.multiple_of(src0 + base(i), 8)
            return pltpu.async_copy(
                inp_hbm.at[pl.ds(s, C), :], bufs[slot], isems[slot]
            )

        def start_out(i):
            slot = i % NBUF
            d = pl.multiple_of(dst0 + base(i), 8)
            return pltpu.async_copy(
                bufs[slot], out_hbm.at[pl.ds(d, C), :], osems[slot]
            )

        in_h = {}
        out_h = {}
        in_h[0] = start_in(0)
        if ITERS > 1:
            in_h[1] = start_in(1)
        for p in range((ITERS + 1) // 2):
            i0, i1 = 2 * p, 2 * p + 1
            in_h[i0].wait()
            if i1 < ITERS:
                in_h[i1].wait()
            out_h[i0] = start_out(i0)
            if i1 < ITERS:
                out_h[i1] = start_out(i1)
            if i0 + 2 < ITERS:
                out_h[i0].wait()
                in_h[i0 + 2] = start_in(i0 + 2)
            if i1 < ITERS and i1 + 2 < ITERS:
                out_h[i1].wait()
                in_h[i1 + 2] = start_in(i1 + 2)
        for i in range(max(0, ITERS - 2), ITERS):
            out_h[i].wait()

    return k(inp)


def kernel(inp, m_splits):
    inp2d = inp.reshape(-1, inp.shape[-1])
    return _unpad(inp2d)


# speedup vs baseline: 1.0505x; 1.0505x over previous
import functools

import jax
import jax.numpy as jnp
from jax import lax
from jax.experimental import pallas as pl
from jax.experimental.pallas import tpu as pltpu
from jax.experimental.pallas import tpu_sc as plsc

NUM_BLOCKS = 8
M = 2000
PM = 2048
D = 2048
NC = 2
NS = 16
W_FULL = 504
W_TAIL = 488
CA = 32
CB = 24
PAIR = CA + CB
ITERS = 2 * (W_FULL // PAIR)


def _chunk(i):
    p, odd = divmod(i, 2)
    size = CB if odd else CA
    base = p * PAIR + (CA if odd else 0)
    bf = min(base, W_FULL - size)
    bt = min(base, W_TAIL - size)
    return bf, bt, size


def _unpad(inp):
    mesh = plsc.VectorSubcoreMesh(core_axis_name="c", subcore_axis_name="s")

    @functools.partial(
        pl.kernel,
        mesh=mesh,
        out_type=jax.ShapeDtypeStruct((NUM_BLOCKS * M, D), jnp.float32),
        scratch_types=(
            [pltpu.VMEM((CA, D), jnp.float32), pltpu.VMEM((CB, D), jnp.float32)]
            + [pltpu.SemaphoreType.DMA] * 4
        ),
    )
    def k(inp_hbm, out_hbm, *scr):
        bufs = scr[:2]
        isems = scr[2:4]
        osems = scr[4:6]
        wid = lax.axis_index("s") * NC + lax.axis_index("c")
        blk = wid // 4
        sub = wid % 4
        off = sub * W_FULL
        src0 = blk * PM + off
        dst0 = blk * M + off
        is_tail = sub == 3

        def base(i):
            bf, bt, _ = _chunk(i)
            if bf == bt:
                return bf
            return jnp.where(is_tail, bt, bf)

        def start_in(i):
            slot = i % 2
            size = _chunk(i)[2]
            s = pl.multiple_of(src0 + base(i), 8)
            return pltpu.async_copy(
                inp_hbm.at[pl.ds(s, size), :], bufs[slot], isems[slot]
            )

        def start_out(i):
            slot = i % 2
            size = _chunk(i)[2]
            d = pl.multiple_of(dst0 + base(i), 8)
            return pltpu.async_copy(
                bufs[slot], out_hbm.at[pl.ds(d, size), :], osems[slot]
            )

        in_h = {0: start_in(0), 1: start_in(1)}
        out_h = {}
        for i in range(ITERS):
            in_h[i].wait()
            out_h[i] = start_out(i)
            if i + 2 < ITERS:
                out_h[i].wait()
                in_h[i + 2] = start_in(i + 2)
        for i in range(ITERS - 2, ITERS):
            out_h[i].wait()

    return k(inp)


def kernel(inp, m_splits):
    inp2d = inp.reshape(-1, inp.shape[-1])
    return _unpad(inp2d)


# speedup vs baseline: 1.0615x; 1.0104x over previous
import functools

import jax
import jax.numpy as jnp
from jax import lax
from jax.experimental import pallas as pl
from jax.experimental.pallas import tpu as pltpu
from jax.experimental.pallas import tpu_sc as plsc

NUM_BLOCKS = 8
M = 2000
PM = 2048
D = 2048
NC = 2
NS = 16
W_FULL = 504
W_TAIL = 488
C = 24
ITERS = W_FULL // C
ROLLED_PAIRS = (ITERS - 3) // 2


def _unpad(inp):
    mesh = plsc.VectorSubcoreMesh(core_axis_name="c", subcore_axis_name="s")

    @functools.partial(
        pl.kernel,
        mesh=mesh,
        out_type=jax.ShapeDtypeStruct((NUM_BLOCKS * M, D), jnp.float32),
        scratch_types=(
            [pltpu.VMEM((C, D), jnp.float32)] * 2
            + [pltpu.SemaphoreType.DMA] * 4
        ),
    )
    def k(inp_hbm, out_hbm, b0, b1, i0, i1, o0, o1):
        bufs = (b0, b1)
        isems = (i0, i1)
        osems = (o0, o1)
        wid = lax.axis_index("s") * NC + lax.axis_index("c")
        blk = wid // 4
        sub = wid % 4
        off = sub * W_FULL
        src0 = blk * PM + off
        dst0 = blk * M + off
        is_tail = sub == 3
        last_base = jnp.where(is_tail, W_TAIL - C, W_FULL - C)

        def start_in(base, slot):
            s = pl.multiple_of(src0 + base, 8)
            return pltpu.async_copy(
                inp_hbm.at[pl.ds(s, C), :], bufs[slot], isems[slot]
            )

        def start_out(base, slot):
            d = pl.multiple_of(dst0 + base, 8)
            return pltpu.async_copy(
                bufs[slot], out_hbm.at[pl.ds(d, C), :], osems[slot]
            )

        def wait_in(slot):
            pltpu.make_async_copy(
                inp_hbm.at[pl.ds(src0, C), :], bufs[slot], isems[slot]
            ).wait()

        def wait_out(slot):
            pltpu.make_async_copy(
                bufs[slot], out_hbm.at[pl.ds(dst0, C), :], osems[slot]
            ).wait()

        start_in(0, 0)
        start_in(C, 1)

        def body(p, carry):
            for slot in (0, 1):
                b = (2 * p + slot) * C
                nb = (2 * p + slot + 2) * C
                wait_in(slot)
                start_out(b, slot)
                wait_out(slot)
                start_in(nb, slot)
            return carry

        lax.fori_loop(0, ROLLED_PAIRS, body, jnp.int32(0))

        wait_in(0)
        start_out(18 * C, 0)
        wait_out(0)
        start_in(last_base, 0)
        wait_in(1)
        start_out(19 * C, 1)
        wait_in(0)
        start_out(last_base, 0)
        wait_out(1)
        wait_out(0)

    return k(inp)


def kernel(inp, m_splits):
    inp2d = inp.reshape(-1, inp.shape[-1])
    return _unpad(inp2d)
